# Initial kernel scaffold; baseline (speedup 1.0000x reference)
import jax
import jax.numpy as jnp
from jax.experimental import pallas as pl


def kernel(indices, table):
    raise NotImplementedError("write your pallas kernel here")


# speedup vs baseline: 1.1030x; 1.1030x over previous
import functools

import jax
import jax.numpy as jnp
from jax import lax
from jax.experimental import pallas as pl
from jax.experimental.pallas import tpu as pltpu
from jax.experimental.pallas import tpu_sc as plsc

EMBED_DIM = 32
NUM_CORES = 2
NUM_SUBCORES = 16
NUM_WORKERS = NUM_CORES * NUM_SUBCORES


def _make_gather(b_total: int, chunk: int):
  b_per_w = b_total // NUM_WORKERS
  n_chunks = b_per_w // chunk
  mesh = plsc.VectorSubcoreMesh(core_axis_name="c", subcore_axis_name="s")

  @functools.partial(
      pl.kernel,
      mesh=mesh,
      compiler_params=pltpu.CompilerParams(use_tc_tiling_on_sc=False),
      out_type=jax.ShapeDtypeStruct((b_total, EMBED_DIM), jnp.float32),
      scratch_types=[
          pltpu.VMEM((chunk,), jnp.int32),
          pltpu.VMEM((chunk, EMBED_DIM), jnp.float32),
          pltpu.SemaphoreType.DMA,
      ],
  )
  def k(table_hbm, idx_hbm, out_hbm, idx_v, rows_v, sem):
    wid = lax.axis_index("s") * NUM_CORES + lax.axis_index("c")
    base = wid * b_per_w

    def body(i, carry):
      off = base + i * chunk
      pltpu.sync_copy(idx_hbm.at[pl.ds(off, chunk)], idx_v)
      pltpu.async_copy(table_hbm.at[idx_v], rows_v, sem).wait()
      pltpu.sync_copy(rows_v, out_hbm.at[pl.ds(off, chunk)])
      return carry

    lax.fori_loop(0, n_chunks, body, 0)

  return k


def kernel(indices, table):
  b, h = indices.shape
  flat_idx = indices.reshape(-1).astype(jnp.int32)
  gather = _make_gather(b * h, chunk=1600)
  out = gather(table, flat_idx)
  return out.reshape(b, h, EMBED_DIM)


# speedup vs baseline: 1.1091x; 1.0056x over previous
import functools

import jax
import jax.numpy as jnp
from jax import lax
from jax.experimental import pallas as pl
from jax.experimental.pallas import tpu as pltpu
from jax.experimental.pallas import tpu_sc as plsc

EMBED_DIM = 32
NUM_CORES = 2
NUM_SUBCORES = 16
NUM_WORKERS = NUM_CORES * NUM_SUBCORES
NBUF = 4
CHUNK = 800


def _make_gather(b_total: int):
  b_per_w = b_total // NUM_WORKERS
  n_chunks = b_per_w // CHUNK
  n_groups = n_chunks // NBUF
  mesh = plsc.VectorSubcoreMesh(core_axis_name="c", subcore_axis_name="s")

  @functools.partial(
      pl.kernel,
      mesh=mesh,
      compiler_params=pltpu.CompilerParams(use_tc_tiling_on_sc=False),
      out_type=jax.ShapeDtypeStruct((b_total, EMBED_DIM), jnp.float32),
      scratch_types=(
          [pltpu.VMEM((b_per_w,), jnp.int32)]
          + [pltpu.VMEM((CHUNK, EMBED_DIM), jnp.float32) for _ in range(NBUF)]
          + [pltpu.SemaphoreType.DMA for _ in range(2 * NBUF)]
      ),
  )
  def k(table_hbm, idx_hbm, out_hbm, idx_v, *bufs_and_sems):
    bufs = bufs_and_sems[:NBUF]
    gsem = bufs_and_sems[NBUF:2 * NBUF]
    ssem = bufs_and_sems[2 * NBUF:]
    wid = lax.axis_index("s") * NUM_CORES + lax.axis_index("c")
    base = wid * b_per_w

    pltpu.sync_copy(idx_hbm.at[pl.ds(base, b_per_w)], idx_v)

    def start_gather(c, b):
      pltpu.async_copy(
          table_hbm.at[idx_v.at[pl.ds(c * CHUNK, CHUNK)]], bufs[b], gsem[b])

    def wait_gather(b):
      pltpu.make_async_copy(
          table_hbm.at[idx_v.at[pl.ds(0, CHUNK)]], bufs[b], gsem[b]).wait()

    def start_store(c, b):
      pltpu.async_copy(
          bufs[b], out_hbm.at[pl.ds(base + c * CHUNK, CHUNK)], ssem[b])

    def wait_store(b):
      pltpu.make_async_copy(
          bufs[b], out_hbm.at[pl.ds(base, CHUNK)], ssem[b]).wait()

    for b in range(NBUF):
      start_gather(b, b)

    def body(j, carry):
      c0 = j * NBUF
      for b in range(NBUF):
        wait_gather(b)
        start_store(c0 + b, b)
      for b in range(NBUF):
        wait_store(b)
        start_gather(c0 + NBUF + b, b)
      return carry

    lax.fori_loop(0, n_groups - 1, body, 0)

    c0 = (n_groups - 1) * NBUF
    for b in range(NBUF):
      wait_gather(b)
      start_store(c0 + b, b)
    for b in range(NBUF):
      wait_store(b)

  return k


def kernel(indices, table):
  b, h = indices.shape
  flat_idx = indices.reshape(-1).astype(jnp.int32)
  gather = _make_gather(b * h)
  out = gather(table, flat_idx)
  return out.reshape(b, h, EMBED_DIM)


# speedup vs baseline: 1.7747x; 1.6001x over previous
import functools

import jax
import jax.numpy as jnp
from jax import lax
from jax.experimental import pallas as pl
from jax.experimental.pallas import tpu as pltpu
from jax.experimental.pallas import tpu_sc as plsc

EMBED_DIM = 32
NUM_CORES = 2
NUM_SUBCORES = 16
NUM_WORKERS = NUM_CORES * NUM_SUBCORES
NBUF = 8


def _make_gather(batch: int, hist: int):
  rows_per_w = batch // NUM_WORKERS
  b_per_w = rows_per_w * hist
  n_groups = rows_per_w // NBUF
  mesh = plsc.VectorSubcoreMesh(core_axis_name="c", subcore_axis_name="s")

  @functools.partial(
      pl.kernel,
      mesh=mesh,
      compiler_params=pltpu.CompilerParams(use_tc_tiling_on_sc=False),
      out_type=jax.ShapeDtypeStruct((batch, hist, EMBED_DIM), jnp.float32),
      scratch_types=(
          [pltpu.VMEM((rows_per_w, hist), jnp.int32)]
          + [pltpu.VMEM((hist, EMBED_DIM), jnp.float32) for _ in range(NBUF)]
          + [pltpu.SemaphoreType.DMA for _ in range(2 * NBUF)]
      ),
  )
  def k(table_hbm, idx_hbm, out_hbm, idx_v, *bufs_and_sems):
    bufs = bufs_and_sems[:NBUF]
    gsem = bufs_and_sems[NBUF:2 * NBUF]
    ssem = bufs_and_sems[2 * NBUF:]
    wid = lax.axis_index("s") * NUM_CORES + lax.axis_index("c")
    row_base = wid * rows_per_w

    pltpu.sync_copy(idx_hbm.at[pl.ds(row_base, rows_per_w), :], idx_v)

    def start_gather(c, b):
      pltpu.async_copy(table_hbm.at[idx_v.at[c]], bufs[b], gsem[b])

    def wait_gather(b):
      pltpu.make_async_copy(
          table_hbm.at[idx_v.at[0]], bufs[b], gsem[b]).wait()

    def start_store(c, b):
      pltpu.async_copy(bufs[b], out_hbm.at[row_base + c], ssem[b])

    def wait_store(b):
      pltpu.make_async_copy(bufs[b], out_hbm.at[row_base], ssem[b]).wait()

    for b in range(NBUF):
      start_gather(b, b)

    def body(j, carry):
      c0 = j * NBUF
      for b in range(NBUF):
        wait_gather(b)
        start_store(c0 + b, b)
      for b in range(NBUF):
        wait_store(b)
        start_gather(c0 + NBUF + b, b)
      return carry

    lax.fori_loop(0, n_groups - 1, body, 0)

    c0 = (n_groups - 1) * NBUF
    for b in range(NBUF):
      wait_gather(b)
      start_store(c0 + b, b)
    for b in range(NBUF):
      wait_store(b)

  return k


def kernel(indices, table):
  b, h = indices.shape
  gather = _make_gather(b, h)
  return gather(table, indices)
